# baseline (device time: 18412 ns/iter reference)
import jax
import jax.numpy as jnp
from jax import lax
from jax.experimental import pallas as pl
from jax.experimental.pallas import tpu as pltpu

N_DEV = 4
B = 2
SQ = 256
SKV = 256
HQ_PER = 4
DH = 64
DM = 512


def kernel(x, Wq, K_ext, V_ext, Wo):
    my = lax.axis_index("i")
    k_loc = lax.dynamic_slice_in_dim(
        K_ext, my * HQ_PER, HQ_PER, axis=2
    ).reshape(B, SKV, HQ_PER * DH).astype(jnp.bfloat16)
    v_loc = lax.dynamic_slice_in_dim(
        V_ext, my * HQ_PER, HQ_PER, axis=2
    ).reshape(B, SKV, HQ_PER * DH).astype(jnp.bfloat16)
    wq_s = (Wq * 0.125).astype(jnp.bfloat16)
    wo_s = Wo.astype(jnp.bfloat16)

    def body(x_hbm, wq_hbm, k_hbm, v_hbm, wo_hbm, out_ref,
             xv, wqv, kv, vv, wov, accv,
             send_a, recv_a, send_b, recv_b,
             load_sems, store_sem,
             send_sems_a, recv_sems_a, send_sems_b, recv_sems_b):
        my_i = lax.axis_index("i")
        left = (my_i - 1) % N_DEV
        right = (my_i + 1) % N_DEV
        partner_a = my_i ^ 1
        partner_b = 3 - my_i

        loads = [
            pltpu.make_async_copy(x_hbm, xv, load_sems.at[0]),
            pltpu.make_async_copy(wq_hbm, wqv, load_sems.at[1]),
            pltpu.make_async_copy(k_hbm, kv, load_sems.at[2]),
            pltpu.make_async_copy(v_hbm, vv, load_sems.at[3]),
            pltpu.make_async_copy(wo_hbm, wov, load_sems.at[4]),
        ]
        for c in loads:
            c.start()

        barrier = pltpu.get_barrier_semaphore()
        for nbr in (left, right):
            pl.semaphore_signal(
                barrier, inc=1,
                device_id=(nbr,), device_id_type=pl.DeviceIdType.MESH,
            )
        pl.semaphore_wait(barrier, 2)

        for c in loads:
            c.wait()

        wq = wqv[...]
        wo = wov[...]

        def softmax_ctx(q, k, v):
            s = lax.dot_general(
                q, k, (((1,), (1,)), ((), ())),
                preferred_element_type=jnp.float32,
            )
            w = jnp.exp(s)
            r = 1.0 / jnp.sum(w, axis=-1, keepdims=True)
            return jnp.dot((w * r).astype(jnp.bfloat16), v,
                           preferred_element_type=jnp.float32)

        def attention_ctx(b):
            xb = xv[b].astype(jnp.bfloat16)
            qf = jnp.dot(xb, wq, preferred_element_type=jnp.float32)
            ctx_blocks = []
            for h in range(HQ_PER):
                qh = qf[:, h * DH:(h + 1) * DH].astype(jnp.bfloat16)
                kh = kv[b][:, h * DH:(h + 1) * DH]
                vh = vv[b][:, h * DH:(h + 1) * DH]
                ctx_a = softmax_ctx(qh[64:192], kh[0:192], vh[0:192])
                qg = jnp.concatenate([qh[0:64], qh[192:256]], axis=0)
                kg = jnp.concatenate([kh[0:64], kh[192:256]], axis=0)
                vg = jnp.concatenate([vh[0:64], vh[192:256]], axis=0)
                ctx_b = softmax_ctx(qg, kg, vg)
                ctx_blocks.append(jnp.concatenate(
                    [ctx_b[0:64], ctx_a, ctx_b[64:128]], axis=0,
                ).astype(jnp.bfloat16))
            return jnp.concatenate(ctx_blocks, axis=1)

        NC = 4
        HS = SQ // NC

        def exchange(phase_send, phase_recv, ssems, rsems, partner, b, c):
            return pltpu.make_async_remote_copy(
                src_ref=phase_send.at[b, pl.ds(c * HS, HS)],
                dst_ref=phase_recv.at[b, pl.ds(c * HS, HS)],
                send_sem=ssems.at[b, c],
                recv_sem=rsems.at[b, c],
                device_id=(partner,),
                device_id_type=pl.DeviceIdType.MESH,
            )

        rdma_1 = {}
        rdma_2 = {}
        for b in range(B):
            ctx_full = attention_ctx(b)
            for half in range(2):
                rs = slice(half * 2 * HS, (half + 1) * 2 * HS)
                acc_h = jnp.dot(ctx_full[rs], wo,
                                preferred_element_type=jnp.float32)
                accv[b, rs] = acc_h
                send_a[b, rs] = acc_h.astype(jnp.bfloat16)
                for c in (2 * half, 2 * half + 1):
                    p1 = partner_a if (b + c) % 2 == 0 else partner_b
                    rdma_1[b, c] = exchange(send_a, recv_a,
                                            send_sems_a, recv_sems_a, p1, b, c)
                    rdma_1[b, c].start()

        for b in range(B):
            for c in range(NC):
                rdma_1[b, c].wait()
                cs = slice(c * HS, (c + 1) * HS)
                pair_sum = accv[b, cs] + recv_a[b, cs].astype(jnp.float32)
                accv[b, cs] = pair_sum
                send_b[b, cs] = pair_sum.astype(jnp.bfloat16)
                p2 = partner_b if (b + c) % 2 == 0 else partner_a
                rdma_2[b, c] = exchange(send_b, recv_b,
                                        send_sems_b, recv_sems_b, p2, b, c)
                rdma_2[b, c].start()

        for b in range(B):
            for c in range(NC):
                rdma_2[b, c].wait()
                cs = slice(c * HS, (c + 1) * HS)
                accv[b, cs] = accv[b, cs] + recv_b[b, cs].astype(jnp.float32)

        store = pltpu.make_async_copy(accv, out_ref, store_sem)
        store.start()
        store.wait()

    comm = pltpu.VMEM((B, SQ, DM), jnp.bfloat16)
    return pl.pallas_call(
        body,
        out_shape=jax.ShapeDtypeStruct((B, SQ, DM), jnp.float32),
        in_specs=[pl.BlockSpec(memory_space=pl.ANY)] * 5,
        out_specs=pl.BlockSpec(memory_space=pl.ANY),
        scratch_shapes=[
            pltpu.VMEM((B, SQ, DM), jnp.float32),
            pltpu.VMEM((DM, HQ_PER * DH), jnp.bfloat16),
            pltpu.VMEM((B, SKV, HQ_PER * DH), jnp.bfloat16),
            pltpu.VMEM((B, SKV, HQ_PER * DH), jnp.bfloat16),
            pltpu.VMEM((HQ_PER * DH, DM), jnp.bfloat16),
            pltpu.VMEM((B, SQ, DM), jnp.float32),
            comm, comm, comm, comm,
            pltpu.SemaphoreType.DMA((5,)),
            pltpu.SemaphoreType.DMA,
            pltpu.SemaphoreType.DMA((B, 4)),
            pltpu.SemaphoreType.DMA((B, 4)),
            pltpu.SemaphoreType.DMA((B, 4)),
            pltpu.SemaphoreType.DMA((B, 4)),
        ],
        compiler_params=pltpu.CompilerParams(collective_id=0),
    )(x, wq_s, k_loc, v_loc, wo_s)


# device time: 16351 ns/iter; 1.1260x vs baseline; 1.1260x over previous
import jax
import jax.numpy as jnp
from jax import lax
from jax.experimental import pallas as pl
from jax.experimental.pallas import tpu as pltpu

N_DEV = 4
B = 2
SQ = 256
SKV = 256
HQ_PER = 4
DH = 64
DM = 512


def kernel(x, Wq, K_ext, V_ext, Wo):
    my = lax.axis_index("i")
    k_loc = lax.dynamic_slice_in_dim(
        K_ext, my * HQ_PER, HQ_PER, axis=2
    ).reshape(B, SKV, HQ_PER * DH).astype(jnp.bfloat16)
    v_loc = lax.dynamic_slice_in_dim(
        V_ext, my * HQ_PER, HQ_PER, axis=2
    ).reshape(B, SKV, HQ_PER * DH).astype(jnp.bfloat16)

    def body(x_hbm, wq_hbm, k_hbm, v_hbm, wo_hbm, out_ref,
             xv, wqv, kv, vv, wov, accv,
             send_a, recv_a, send_b, recv_b,
             load_sems, store_sem,
             send_sems_a, recv_sems_a, send_sems_b, recv_sems_b):
        my_i = lax.axis_index("i")
        left = (my_i - 1) % N_DEV
        right = (my_i + 1) % N_DEV
        partner_a = my_i ^ 1
        partner_b = 3 - my_i

        loads = [
            pltpu.make_async_copy(x_hbm, xv, load_sems.at[0]),
            pltpu.make_async_copy(wq_hbm, wqv, load_sems.at[1]),
            pltpu.make_async_copy(k_hbm, kv, load_sems.at[2]),
            pltpu.make_async_copy(v_hbm, vv, load_sems.at[3]),
            pltpu.make_async_copy(wo_hbm, wov, load_sems.at[4]),
        ]
        for c in loads:
            c.start()

        barrier = pltpu.get_barrier_semaphore()
        for nbr in (left, right):
            pl.semaphore_signal(
                barrier, inc=1,
                device_id=(nbr,), device_id_type=pl.DeviceIdType.MESH,
            )
        pl.semaphore_wait(barrier, 2)

        for c in loads:
            c.wait()

        wq = (wqv[...] * 0.125).astype(jnp.bfloat16)
        wo = wov[...].astype(jnp.bfloat16)

        def softmax_ctx(q, k, v):
            s = lax.dot_general(
                q, k, (((1,), (1,)), ((), ())),
                preferred_element_type=jnp.float32,
            )
            w = jnp.exp(s)
            r = 1.0 / jnp.sum(w, axis=-1, keepdims=True)
            return jnp.dot((w * r).astype(jnp.bfloat16), v,
                           preferred_element_type=jnp.float32)

        def attention_ctx(b):
            xb = xv[b].astype(jnp.bfloat16)
            qf = jnp.dot(xb, wq, preferred_element_type=jnp.float32)
            ctx_blocks = []
            for h in range(HQ_PER):
                qh = qf[:, h * DH:(h + 1) * DH].astype(jnp.bfloat16)
                kh = kv[b][:, h * DH:(h + 1) * DH]
                vh = vv[b][:, h * DH:(h + 1) * DH]
                ctx_a = softmax_ctx(qh[64:192], kh[0:192], vh[0:192])
                qg = jnp.concatenate([qh[0:64], qh[192:256]], axis=0)
                kg = jnp.concatenate([kh[0:64], kh[192:256]], axis=0)
                vg = jnp.concatenate([vh[0:64], vh[192:256]], axis=0)
                ctx_b = softmax_ctx(qg, kg, vg)
                ctx_blocks.append(jnp.concatenate(
                    [ctx_b[0:64], ctx_a, ctx_b[64:128]], axis=0,
                ).astype(jnp.bfloat16))
            return jnp.concatenate(ctx_blocks, axis=1)

        NC = 4
        HS = SQ // NC

        def exchange(phase_send, phase_recv, ssems, rsems, partner, b, c):
            return pltpu.make_async_remote_copy(
                src_ref=phase_send.at[b, pl.ds(c * HS, HS)],
                dst_ref=phase_recv.at[b, pl.ds(c * HS, HS)],
                send_sem=ssems.at[b, c],
                recv_sem=rsems.at[b, c],
                device_id=(partner,),
                device_id_type=pl.DeviceIdType.MESH,
            )

        rdma_1 = {}
        rdma_2 = {}
        for b in range(B):
            ctx_full = attention_ctx(b)
            for half in range(2):
                rs = slice(half * 2 * HS, (half + 1) * 2 * HS)
                acc_h = jnp.dot(ctx_full[rs], wo,
                                preferred_element_type=jnp.float32)
                accv[b, rs] = acc_h
                send_a[b, rs] = acc_h.astype(jnp.bfloat16)
                for c in (2 * half, 2 * half + 1):
                    p1 = partner_a if (b + c) % 2 == 0 else partner_b
                    rdma_1[b, c] = exchange(send_a, recv_a,
                                            send_sems_a, recv_sems_a, p1, b, c)
                    rdma_1[b, c].start()

        for b in range(B):
            for c in range(NC):
                rdma_1[b, c].wait()
                cs = slice(c * HS, (c + 1) * HS)
                pair_sum = accv[b, cs] + recv_a[b, cs].astype(jnp.float32)
                accv[b, cs] = pair_sum
                send_b[b, cs] = pair_sum.astype(jnp.bfloat16)
                p2 = partner_b if (b + c) % 2 == 0 else partner_a
                rdma_2[b, c] = exchange(send_b, recv_b,
                                        send_sems_b, recv_sems_b, p2, b, c)
                rdma_2[b, c].start()

        for b in range(B):
            for c in range(NC):
                rdma_2[b, c].wait()
                cs = slice(c * HS, (c + 1) * HS)
                accv[b, cs] = accv[b, cs] + recv_b[b, cs].astype(jnp.float32)

        store = pltpu.make_async_copy(accv, out_ref, store_sem)
        store.start()
        store.wait()

    comm = pltpu.VMEM((B, SQ, DM), jnp.bfloat16)
    return pl.pallas_call(
        body,
        out_shape=jax.ShapeDtypeStruct((B, SQ, DM), jnp.float32),
        in_specs=[pl.BlockSpec(memory_space=pl.ANY)] * 5,
        out_specs=pl.BlockSpec(memory_space=pl.ANY),
        scratch_shapes=[
            pltpu.VMEM((B, SQ, DM), jnp.float32),
            pltpu.VMEM((DM, HQ_PER * DH), jnp.float32),
            pltpu.VMEM((B, SKV, HQ_PER * DH), jnp.bfloat16),
            pltpu.VMEM((B, SKV, HQ_PER * DH), jnp.bfloat16),
            pltpu.VMEM((HQ_PER * DH, DM), jnp.float32),
            pltpu.VMEM((B, SQ, DM), jnp.float32),
            comm, comm, comm, comm,
            pltpu.SemaphoreType.DMA((5,)),
            pltpu.SemaphoreType.DMA,
            pltpu.SemaphoreType.DMA((B, 4)),
            pltpu.SemaphoreType.DMA((B, 4)),
            pltpu.SemaphoreType.DMA((B, 4)),
            pltpu.SemaphoreType.DMA((B, 4)),
        ],
        compiler_params=pltpu.CompilerParams(collective_id=0),
    )(x, Wq, k_loc, v_loc, Wo)
